# baseline (speedup 1.0000x reference)
import jax
import jax.numpy as jnp
from jax import lax
from jax.experimental import pallas as pl
from jax.experimental.pallas import tpu as pltpu
from jax.experimental.pallas import tpu_sc as plsc

H = 32
R = 4
HR = H * R
LANES = 16
C = 128
NBUF = 2
TBLK = 16384


def _quarter_width(V):
    return (V // R + 127) // 128 * 128


def _tc_quadpack(V):
    V4 = _quarter_width(V)
    W3 = V - 3 * V4
    grid = pl.cdiv(V4, TBLK)
    g3f = W3 // TBLK
    edge = W3 - g3f * TBLK

    def body(tt_ref, o_ref, xb0, xb1, xbe, sem):
        g = pl.program_id(0)
        xbufs = (xb0, xb1)

        def start(gi, xb):
            for q in range(R - 1):
                pltpu.make_async_copy(
                    tt_ref.at[:, pl.ds(q * V4 + gi * TBLK, TBLK)],
                    xb.at[q],
                    sem,
                ).start()

            @pl.when(gi < g3f)
            def _():
                pltpu.make_async_copy(
                    tt_ref.at[:, pl.ds(3 * V4 + gi * TBLK, TBLK)],
                    xb.at[3],
                    sem,
                ).start()

            if edge:
                @pl.when(gi == g3f)
                def _():
                    pltpu.make_async_copy(
                        tt_ref.at[:, pl.ds(3 * V4 + g3f * TBLK, edge)],
                        xbe,
                        sem,
                    ).start()

        def wait_for(gi, xb):
            for q in range(R - 1):
                pltpu.make_async_copy(
                    tt_ref.at[:, pl.ds(0, TBLK)], xb.at[q], sem
                ).wait()

            @pl.when(gi < g3f)
            def _():
                pltpu.make_async_copy(
                    tt_ref.at[:, pl.ds(0, TBLK)], xb.at[3], sem
                ).wait()

            if edge:
                @pl.when(gi == g3f)
                def _():
                    pltpu.make_async_copy(
                        tt_ref.at[:, pl.ds(3 * V4 + g3f * TBLK, edge)],
                        xbe,
                        sem,
                    ).wait()

        @pl.when(g == 0)
        def _():
            start(0, xbufs[0])

        def phase(par):
            @pl.when(lax.rem(g, 2) == par)
            def _():
                xb = xbufs[par]
                nxt = xbufs[1 - par]

                @pl.when(g + 1 < grid)
                def _():
                    start(g + 1, nxt)

                wait_for(g, xb)
                for q in range(R):
                    o_ref[:, q * H:(q + 1) * H] = xb[q].T
                if edge:
                    @pl.when(g == g3f)
                    def _():
                        o_ref[pl.ds(0, edge), (R - 1) * H:] = xbe[...].T

        phase(0)
        phase(1)

    return pl.pallas_call(
        body,
        grid=(grid,),
        in_specs=[pl.BlockSpec(memory_space=pl.ANY)],
        out_specs=pl.BlockSpec((TBLK, HR), lambda g: (g, 0)),
        out_shape=jax.ShapeDtypeStruct((V4, HR), jnp.float32),
        scratch_shapes=[
            pltpu.VMEM((R, H, TBLK), jnp.float32),
            pltpu.VMEM((R, H, TBLK), jnp.float32),
            pltpu.VMEM((H, edge if edge else 128), jnp.float32),
            pltpu.SemaphoreType.DMA,
        ],
    )


def _sc_kernel(B, b_per_w, num_cores, V4):
    mesh = plsc.VectorSubcoreMesh(core_axis_name="c", subcore_axis_name="s")
    n_chunks = b_per_w // C
    n_outer = n_chunks // NBUF

    def body(
        ids_hbm, table_hbm, w_hbm, out_hbm,
        idx_v, idx4_v, in_v0, in_v1, out_v0, out_v1, w_v,
        gsem0, gsem1, ssem0, ssem1,
    ):
        in_vs = [in_v0, in_v1]
        out_vs = [out_v0, out_v1]
        gsems = [gsem0, gsem1]
        ssems = [ssem0, ssem1]
        wid = lax.axis_index("s") * num_cores + lax.axis_index("c")
        base = wid * b_per_w
        pltpu.sync_copy(
            ids_hbm.at[pl.ds(base, b_per_w)], idx_v.at[pl.ds(0, b_per_w)]
        )
        pltpu.sync_copy(w_hbm, w_v)
        w_vecs = [w_v[pl.ds(LANES * k, LANES)] for k in range(HR // LANES)]
        rep = lax.iota(jnp.int32, LANES) // R
        cidx = [rep + k * R for k in range(HR // LANES)]

        @plsc.parallel_loop(0, b_per_w // LANES, unroll=8)
        def quad_body(i):
            v = idx_v[pl.ds(i * LANES, LANES)]
            q = (
                (v >= V4).astype(jnp.int32)
                + (v >= 2 * V4).astype(jnp.int32)
                + (v >= 3 * V4).astype(jnp.int32)
            )
            idx4_v[pl.ds(i * LANES, LANES)] = (v - q * V4) * R + q

        for b in range(NBUF):
            pltpu.async_copy(
                table_hbm.at[idx4_v.at[pl.ds(b * C, C)]], in_vs[b], gsems[b]
            )

        def outer(i, _):
            for b in range(NBUF):
                ch = i * NBUF + b
                in_v, out_v = in_vs[b], out_vs[b]
                pltpu.make_async_copy(
                    table_hbm.at[pl.ds(0, C)], in_v, gsems[b]
                ).wait()

                @pl.when(i >= 1)
                def _():
                    pltpu.make_async_copy(
                        out_v, out_hbm.at[pl.ds(base, C)], ssems[b]
                    ).wait()

                @plsc.parallel_loop(0, C, unroll=4)
                def row_body(row):
                    rsp = jnp.full((LANES,), row, dtype=jnp.int32)
                    for k in range(HR // LANES):
                        g = plsc.load_gather(in_v, [rsp, cidx[k]])
                        out_v[row, pl.ds(LANES * k, LANES)] = g * w_vecs[k]

                pltpu.async_copy(
                    out_v, out_hbm.at[pl.ds(base + ch * C, C)], ssems[b]
                )

                @pl.when(ch + NBUF < n_chunks)
                def _():
                    pltpu.async_copy(
                        table_hbm.at[idx4_v.at[pl.ds((ch + NBUF) * C, C)]],
                        in_v,
                        gsems[b],
                    )
            return 0

        lax.fori_loop(0, n_outer, outer, 0)
        for b in range(NBUF):
            pltpu.make_async_copy(
                out_vs[b], out_hbm.at[pl.ds(base, C)], ssems[b]
            ).wait()

    return pl.kernel(
        body,
        out_type=jax.ShapeDtypeStruct((B, HR), jnp.float32),
        mesh=mesh,
        scratch_types=[
            pltpu.VMEM((b_per_w,), jnp.int32),
            pltpu.VMEM((b_per_w,), jnp.int32),
            pltpu.VMEM((C, H), jnp.float32),
            pltpu.VMEM((C, H), jnp.float32),
            pltpu.VMEM((C, HR), jnp.float32),
            pltpu.VMEM((C, HR), jnp.float32),
            pltpu.VMEM((HR,), jnp.float32),
            pltpu.SemaphoreType.DMA,
            pltpu.SemaphoreType.DMA,
            pltpu.SemaphoreType.DMA,
            pltpu.SemaphoreType.DMA,
        ],
        compiler_params=pltpu.CompilerParams(
            use_tc_tiling_on_sc=False, needs_layout_passes=False
        ),
    )


def kernel(input_ids, table, w_up):
    bs, l = input_ids.shape
    B = bs * l
    V = table.shape[0]
    info = plsc.get_sparse_core_info()
    nw = info.num_cores * info.num_subcores
    b_per_w = B // nw
    assert b_per_w * nw == B and b_per_w % (C * NBUF) == 0
    assert V % 8 == 0 and 3 * _quarter_width(V) < V
    ids2 = input_ids.T.reshape(B).astype(jnp.int32)
    tt = table.T
    V4 = _quarter_width(V)
    table_lin = _tc_quadpack(V)(tt).reshape(R * V4, H)
    out2 = _sc_kernel(B, b_per_w, info.num_cores, V4)(ids2, table_lin, w_up)
    return out2.reshape(l, bs, HR).transpose(1, 0, 2)


# speedup vs baseline: 1.0227x; 1.0227x over previous
import jax
import jax.numpy as jnp
from jax import lax
from jax.experimental import pallas as pl
from jax.experimental.pallas import tpu as pltpu
from jax.experimental.pallas import tpu_sc as plsc

H = 32
R = 4
HR = H * R
LANES = 16
C = 128
NBUF = 2
TBLK = 8192


def _quarter_width(V):
    return (V // R + 127) // 128 * 128


def _tc_quadpack(V):
    V4 = _quarter_width(V)
    W3 = V - 3 * V4
    grid = pl.cdiv(V4, TBLK)
    g3f = W3 // TBLK
    edge = W3 - g3f * TBLK

    def body(tt_ref, o_ref, xb0, xb1, xbe, sem):
        g = pl.program_id(0)
        xbufs = (xb0, xb1)

        def start(gi, xb):
            for q in range(R - 1):
                pltpu.make_async_copy(
                    tt_ref.at[:, pl.ds(q * V4 + gi * TBLK, TBLK)],
                    xb.at[q],
                    sem,
                ).start()

            @pl.when(gi < g3f)
            def _():
                pltpu.make_async_copy(
                    tt_ref.at[:, pl.ds(3 * V4 + gi * TBLK, TBLK)],
                    xb.at[3],
                    sem,
                ).start()

            if edge:
                @pl.when(gi == g3f)
                def _():
                    pltpu.make_async_copy(
                        tt_ref.at[:, pl.ds(3 * V4 + g3f * TBLK, edge)],
                        xbe,
                        sem,
                    ).start()

        def wait_for(gi, xb):
            for q in range(R - 1):
                pltpu.make_async_copy(
                    tt_ref.at[:, pl.ds(0, TBLK)], xb.at[q], sem
                ).wait()

            @pl.when(gi < g3f)
            def _():
                pltpu.make_async_copy(
                    tt_ref.at[:, pl.ds(0, TBLK)], xb.at[3], sem
                ).wait()

            if edge:
                @pl.when(gi == g3f)
                def _():
                    pltpu.make_async_copy(
                        tt_ref.at[:, pl.ds(3 * V4 + g3f * TBLK, edge)],
                        xbe,
                        sem,
                    ).wait()

        @pl.when(g == 0)
        def _():
            start(0, xbufs[0])

        def phase(par):
            @pl.when(lax.rem(g, 2) == par)
            def _():
                xb = xbufs[par]
                nxt = xbufs[1 - par]

                @pl.when(g + 1 < grid)
                def _():
                    start(g + 1, nxt)

                wait_for(g, xb)
                o_ref[...] = jnp.concatenate(
                    [xb[q].T for q in range(R)], axis=1
                )
                if edge:
                    @pl.when(g == g3f)
                    def _():
                        o_ref[pl.ds(0, edge), (R - 1) * H:] = xbe[...].T

        phase(0)
        phase(1)

    return pl.pallas_call(
        body,
        grid=(grid,),
        in_specs=[pl.BlockSpec(memory_space=pl.ANY)],
        out_specs=pl.BlockSpec((TBLK, HR), lambda g: (g, 0)),
        out_shape=jax.ShapeDtypeStruct((V4, HR), jnp.float32),
        scratch_shapes=[
            pltpu.VMEM((R, H, TBLK), jnp.float32),
            pltpu.VMEM((R, H, TBLK), jnp.float32),
            pltpu.VMEM((H, edge if edge else 128), jnp.float32),
            pltpu.SemaphoreType.DMA,
        ],
    )


def _sc_kernel(B, b_per_w, num_cores, V4):
    mesh = plsc.VectorSubcoreMesh(core_axis_name="c", subcore_axis_name="s")
    n_chunks = b_per_w // C
    n_outer = n_chunks // NBUF

    def body(
        ids_hbm, table_hbm, w_hbm, out_hbm,
        idx_v, idx4_v, in_v0, in_v1, out_v0, out_v1, w_v,
        gsem0, gsem1, ssem0, ssem1,
    ):
        in_vs = [in_v0, in_v1]
        out_vs = [out_v0, out_v1]
        gsems = [gsem0, gsem1]
        ssems = [ssem0, ssem1]
        wid = lax.axis_index("s") * num_cores + lax.axis_index("c")
        base = wid * b_per_w
        pltpu.sync_copy(
            ids_hbm.at[pl.ds(base, b_per_w)], idx_v.at[pl.ds(0, b_per_w)]
        )
        pltpu.sync_copy(w_hbm, w_v)
        w_vecs = [w_v[pl.ds(LANES * k, LANES)] for k in range(HR // LANES)]
        rep = lax.iota(jnp.int32, LANES) // R
        cidx = [rep + k * R for k in range(HR // LANES)]

        @plsc.parallel_loop(0, b_per_w // LANES, unroll=8)
        def quad_body(i):
            v = idx_v[pl.ds(i * LANES, LANES)]
            q = (
                (v >= V4).astype(jnp.int32)
                + (v >= 2 * V4).astype(jnp.int32)
                + (v >= 3 * V4).astype(jnp.int32)
            )
            idx4_v[pl.ds(i * LANES, LANES)] = (v - q * V4) * R + q

        for b in range(NBUF):
            pltpu.async_copy(
                table_hbm.at[idx4_v.at[pl.ds(b * C, C)]], in_vs[b], gsems[b]
            )

        def outer(i, _):
            for b in range(NBUF):
                ch = i * NBUF + b
                in_v, out_v = in_vs[b], out_vs[b]
                pltpu.make_async_copy(
                    table_hbm.at[pl.ds(0, C)], in_v, gsems[b]
                ).wait()

                @pl.when(i >= 1)
                def _():
                    pltpu.make_async_copy(
                        out_v, out_hbm.at[pl.ds(base, C)], ssems[b]
                    ).wait()

                @plsc.parallel_loop(0, C, unroll=4)
                def row_body(row):
                    rsp = jnp.full((LANES,), row, dtype=jnp.int32)
                    for k in range(HR // LANES):
                        g = plsc.load_gather(in_v, [rsp, cidx[k]])
                        out_v[row, pl.ds(LANES * k, LANES)] = g * w_vecs[k]

                pltpu.async_copy(
                    out_v, out_hbm.at[pl.ds(base + ch * C, C)], ssems[b]
                )

                @pl.when(ch + NBUF < n_chunks)
                def _():
                    pltpu.async_copy(
                        table_hbm.at[idx4_v.at[pl.ds((ch + NBUF) * C, C)]],
                        in_v,
                        gsems[b],
                    )
            return 0

        lax.fori_loop(0, n_outer, outer, 0)
        for b in range(NBUF):
            pltpu.make_async_copy(
                out_vs[b], out_hbm.at[pl.ds(base, C)], ssems[b]
            ).wait()

    return pl.kernel(
        body,
        out_type=jax.ShapeDtypeStruct((B, HR), jnp.float32),
        mesh=mesh,
        scratch_types=[
            pltpu.VMEM((b_per_w,), jnp.int32),
            pltpu.VMEM((b_per_w,), jnp.int32),
            pltpu.VMEM((C, H), jnp.float32),
            pltpu.VMEM((C, H), jnp.float32),
            pltpu.VMEM((C, HR), jnp.float32),
            pltpu.VMEM((C, HR), jnp.float32),
            pltpu.VMEM((HR,), jnp.float32),
            pltpu.SemaphoreType.DMA,
            pltpu.SemaphoreType.DMA,
            pltpu.SemaphoreType.DMA,
            pltpu.SemaphoreType.DMA,
        ],
        compiler_params=pltpu.CompilerParams(
            use_tc_tiling_on_sc=False, needs_layout_passes=False
        ),
    )


def kernel(input_ids, table, w_up):
    bs, l = input_ids.shape
    B = bs * l
    V = table.shape[0]
    info = plsc.get_sparse_core_info()
    nw = info.num_cores * info.num_subcores
    b_per_w = B // nw
    assert b_per_w * nw == B and b_per_w % (C * NBUF) == 0
    assert V % 8 == 0 and 3 * _quarter_width(V) < V
    ids2 = input_ids.T.reshape(B).astype(jnp.int32)
    tt = table.T
    V4 = _quarter_width(V)
    table_lin = _tc_quadpack(V)(tt).reshape(R * V4, H)
    out2 = _sc_kernel(B, b_per_w, info.num_cores, V4)(ids2, table_lin, w_up)
    return out2.reshape(l, bs, HR).transpose(1, 0, 2)


# speedup vs baseline: 2.0355x; 1.9903x over previous
import jax
import jax.numpy as jnp
from jax import lax
from jax.experimental import pallas as pl
from jax.experimental.pallas import tpu as pltpu
from jax.experimental.pallas import tpu_sc as plsc

H = 32
R = 4
HR = H * R
LANES = 16
C = 128
NBUF = 2
TBLK = 8192


def _quarter_width(V):
    return (V // R + 127) // 128 * 128


def _tc_quadpack(V):
    V4 = _quarter_width(V)
    W3 = V - 3 * V4
    grid = pl.cdiv(V4, TBLK)
    g3f = W3 // TBLK
    edge = W3 - g3f * TBLK

    def body(tt_ref, o_ref, xb0, xb1, xbe, sem):
        g = pl.program_id(0)
        xbufs = (xb0, xb1)

        def start(gi, xb):
            for q in range(R - 1):
                pltpu.make_async_copy(
                    tt_ref.at[:, pl.ds(q * V4 + gi * TBLK, TBLK)],
                    xb.at[pl.ds(q * H, H), :],
                    sem,
                ).start()

            @pl.when(gi < g3f)
            def _():
                pltpu.make_async_copy(
                    tt_ref.at[:, pl.ds(3 * V4 + gi * TBLK, TBLK)],
                    xb.at[pl.ds(3 * H, H), :],
                    sem,
                ).start()

            if edge:
                @pl.when(gi == g3f)
                def _():
                    pltpu.make_async_copy(
                        tt_ref.at[:, pl.ds(3 * V4 + g3f * TBLK, edge)],
                        xbe,
                        sem,
                    ).start()

        def wait_for(gi, xb):
            for q in range(R - 1):
                pltpu.make_async_copy(
                    tt_ref.at[:, pl.ds(0, TBLK)],
                    xb.at[pl.ds(q * H, H), :],
                    sem,
                ).wait()

            @pl.when(gi < g3f)
            def _():
                pltpu.make_async_copy(
                    tt_ref.at[:, pl.ds(0, TBLK)],
                    xb.at[pl.ds(3 * H, H), :],
                    sem,
                ).wait()

            if edge:
                @pl.when(gi == g3f)
                def _():
                    pltpu.make_async_copy(
                        tt_ref.at[:, pl.ds(3 * V4 + g3f * TBLK, edge)],
                        xbe,
                        sem,
                    ).wait()

        @pl.when(g == 0)
        def _():
            start(0, xbufs[0])

        def phase(par):
            @pl.when(lax.rem(g, 2) == par)
            def _():
                xb = xbufs[par]
                nxt = xbufs[1 - par]

                @pl.when(g + 1 < grid)
                def _():
                    start(g + 1, nxt)

                wait_for(g, xb)
                o_ref[...] = xb[...].T
                if edge:
                    @pl.when(g == g3f)
                    def _():
                        o_ref[pl.ds(0, edge), (R - 1) * H:] = xbe[...].T

        phase(0)
        phase(1)

    return pl.pallas_call(
        body,
        grid=(grid,),
        in_specs=[pl.BlockSpec(memory_space=pl.ANY)],
        out_specs=pl.BlockSpec((TBLK, HR), lambda g: (g, 0)),
        out_shape=jax.ShapeDtypeStruct((V4, HR), jnp.float32),
        scratch_shapes=[
            pltpu.VMEM((HR, TBLK), jnp.float32),
            pltpu.VMEM((HR, TBLK), jnp.float32),
            pltpu.VMEM((H, edge if edge else 128), jnp.float32),
            pltpu.SemaphoreType.DMA,
        ],
    )


def _sc_kernel(B, b_per_w, num_cores, V4):
    mesh = plsc.VectorSubcoreMesh(core_axis_name="c", subcore_axis_name="s")
    n_chunks = b_per_w // C
    n_outer = n_chunks // NBUF

    def body(
        ids_hbm, table_hbm, w_hbm, out_hbm,
        idx_v, idx4_v, in_v0, in_v1, out_v0, out_v1, w_v,
        gsem0, gsem1, ssem0, ssem1,
    ):
        in_vs = [in_v0, in_v1]
        out_vs = [out_v0, out_v1]
        gsems = [gsem0, gsem1]
        ssems = [ssem0, ssem1]
        wid = lax.axis_index("s") * num_cores + lax.axis_index("c")
        base = wid * b_per_w
        pltpu.sync_copy(
            ids_hbm.at[pl.ds(base, b_per_w)], idx_v.at[pl.ds(0, b_per_w)]
        )
        pltpu.sync_copy(w_hbm, w_v)
        w_vecs = [w_v[pl.ds(LANES * k, LANES)] for k in range(HR // LANES)]
        rep = lax.iota(jnp.int32, LANES) // R
        cidx = [rep + k * R for k in range(HR // LANES)]

        @plsc.parallel_loop(0, b_per_w // LANES, unroll=8)
        def quad_body(i):
            v = idx_v[pl.ds(i * LANES, LANES)]
            q = (
                (v >= V4).astype(jnp.int32)
                + (v >= 2 * V4).astype(jnp.int32)
                + (v >= 3 * V4).astype(jnp.int32)
            )
            idx4_v[pl.ds(i * LANES, LANES)] = (v - q * V4) * R + q

        for b in range(NBUF):
            pltpu.async_copy(
                table_hbm.at[idx4_v.at[pl.ds(b * C, C)]], in_vs[b], gsems[b]
            )

        def outer(i, _):
            for b in range(NBUF):
                ch = i * NBUF + b
                in_v, out_v = in_vs[b], out_vs[b]
                pltpu.make_async_copy(
                    table_hbm.at[pl.ds(0, C)], in_v, gsems[b]
                ).wait()

                @pl.when(i >= 1)
                def _():
                    pltpu.make_async_copy(
                        out_v, out_hbm.at[pl.ds(base, C)], ssems[b]
                    ).wait()

                @plsc.parallel_loop(0, C, unroll=4)
                def row_body(row):
                    rsp = jnp.full((LANES,), row, dtype=jnp.int32)
                    for k in range(HR // LANES):
                        g = plsc.load_gather(in_v, [rsp, cidx[k]])
                        out_v[row, pl.ds(LANES * k, LANES)] = g * w_vecs[k]

                pltpu.async_copy(
                    out_v, out_hbm.at[pl.ds(base + ch * C, C)], ssems[b]
                )

                @pl.when(ch + NBUF < n_chunks)
                def _():
                    pltpu.async_copy(
                        table_hbm.at[idx4_v.at[pl.ds((ch + NBUF) * C, C)]],
                        in_v,
                        gsems[b],
                    )
            return 0

        lax.fori_loop(0, n_outer, outer, 0)
        for b in range(NBUF):
            pltpu.make_async_copy(
                out_vs[b], out_hbm.at[pl.ds(base, C)], ssems[b]
            ).wait()

    return pl.kernel(
        body,
        out_type=jax.ShapeDtypeStruct((B, HR), jnp.float32),
        mesh=mesh,
        scratch_types=[
            pltpu.VMEM((b_per_w,), jnp.int32),
            pltpu.VMEM((b_per_w,), jnp.int32),
            pltpu.VMEM((C, H), jnp.float32),
            pltpu.VMEM((C, H), jnp.float32),
            pltpu.VMEM((C, HR), jnp.float32),
            pltpu.VMEM((C, HR), jnp.float32),
            pltpu.VMEM((HR,), jnp.float32),
            pltpu.SemaphoreType.DMA,
            pltpu.SemaphoreType.DMA,
            pltpu.SemaphoreType.DMA,
            pltpu.SemaphoreType.DMA,
        ],
        compiler_params=pltpu.CompilerParams(
            use_tc_tiling_on_sc=False, needs_layout_passes=False
        ),
    )


def kernel(input_ids, table, w_up):
    bs, l = input_ids.shape
    B = bs * l
    V = table.shape[0]
    info = plsc.get_sparse_core_info()
    nw = info.num_cores * info.num_subcores
    b_per_w = B // nw
    assert b_per_w * nw == B and b_per_w % (C * NBUF) == 0
    assert V % 8 == 0 and 3 * _quarter_width(V) < V
    ids2 = input_ids.T.reshape(B).astype(jnp.int32)
    tt = table.T
    V4 = _quarter_width(V)
    table_lin = _tc_quadpack(V)(tt).reshape(R * V4, H)
    out2 = _sc_kernel(B, b_per_w, info.num_cores, V4)(ids2, table_lin, w_up)
    return out2.reshape(l, bs, HR).transpose(1, 0, 2)


# speedup vs baseline: 2.0531x; 1.0086x over previous
import jax
import jax.numpy as jnp
from jax import lax
from jax.experimental import pallas as pl
from jax.experimental.pallas import tpu as pltpu
from jax.experimental.pallas import tpu_sc as plsc

H = 32
R = 4
HR = H * R
LANES = 16
C = 128
NBUF = 2
TBLK = 16384


def _quarter_width(V):
    return (V // R + 127) // 128 * 128


def _tc_quadpack(V):
    V4 = _quarter_width(V)
    W3 = V - 3 * V4
    grid = pl.cdiv(V4, TBLK)
    g3f = W3 // TBLK
    edge = W3 - g3f * TBLK

    def body(tt_ref, o_ref, xb0, xb1, xbe, sem):
        g = pl.program_id(0)
        xbufs = (xb0, xb1)

        def start(gi, xb):
            for q in range(R - 1):
                pltpu.make_async_copy(
                    tt_ref.at[:, pl.ds(q * V4 + gi * TBLK, TBLK)],
                    xb.at[pl.ds(q * H, H), :],
                    sem,
                ).start()

            @pl.when(gi < g3f)
            def _():
                pltpu.make_async_copy(
                    tt_ref.at[:, pl.ds(3 * V4 + gi * TBLK, TBLK)],
                    xb.at[pl.ds(3 * H, H), :],
                    sem,
                ).start()

            if edge:
                @pl.when(gi == g3f)
                def _():
                    pltpu.make_async_copy(
                        tt_ref.at[:, pl.ds(3 * V4 + g3f * TBLK, edge)],
                        xbe,
                        sem,
                    ).start()

        def wait_for(gi, xb):
            for q in range(R - 1):
                pltpu.make_async_copy(
                    tt_ref.at[:, pl.ds(0, TBLK)],
                    xb.at[pl.ds(q * H, H), :],
                    sem,
                ).wait()

            @pl.when(gi < g3f)
            def _():
                pltpu.make_async_copy(
                    tt_ref.at[:, pl.ds(0, TBLK)],
                    xb.at[pl.ds(3 * H, H), :],
                    sem,
                ).wait()

            if edge:
                @pl.when(gi == g3f)
                def _():
                    pltpu.make_async_copy(
                        tt_ref.at[:, pl.ds(3 * V4 + g3f * TBLK, edge)],
                        xbe,
                        sem,
                    ).wait()

        @pl.when(g == 0)
        def _():
            start(0, xbufs[0])

        def phase(par):
            @pl.when(lax.rem(g, 2) == par)
            def _():
                xb = xbufs[par]
                nxt = xbufs[1 - par]

                @pl.when(g + 1 < grid)
                def _():
                    start(g + 1, nxt)

                wait_for(g, xb)
                o_ref[...] = xb[...].T
                if edge:
                    @pl.when(g == g3f)
                    def _():
                        o_ref[pl.ds(0, edge), (R - 1) * H:] = xbe[...].T

        phase(0)
        phase(1)

    return pl.pallas_call(
        body,
        grid=(grid,),
        in_specs=[pl.BlockSpec(memory_space=pl.ANY)],
        out_specs=pl.BlockSpec((TBLK, HR), lambda g: (g, 0)),
        out_shape=jax.ShapeDtypeStruct((V4, HR), jnp.float32),
        scratch_shapes=[
            pltpu.VMEM((HR, TBLK), jnp.float32),
            pltpu.VMEM((HR, TBLK), jnp.float32),
            pltpu.VMEM((H, edge if edge else 128), jnp.float32),
            pltpu.SemaphoreType.DMA,
        ],
    )


def _sc_kernel(B, b_per_w, num_cores, V4):
    mesh = plsc.VectorSubcoreMesh(core_axis_name="c", subcore_axis_name="s")
    n_chunks = b_per_w // C
    n_outer = n_chunks // NBUF

    def body(
        ids_hbm, table_hbm, w_hbm, out_hbm,
        idx_v, idx4_v, in_v0, in_v1, out_v0, out_v1, w_v,
        gsem0, gsem1, ssem0, ssem1,
    ):
        in_vs = [in_v0, in_v1]
        out_vs = [out_v0, out_v1]
        gsems = [gsem0, gsem1]
        ssems = [ssem0, ssem1]
        wid = lax.axis_index("s") * num_cores + lax.axis_index("c")
        base = wid * b_per_w
        pltpu.sync_copy(
            ids_hbm.at[pl.ds(base, b_per_w)], idx_v.at[pl.ds(0, b_per_w)]
        )
        pltpu.sync_copy(w_hbm, w_v)
        w_vecs = [w_v[pl.ds(LANES * k, LANES)] for k in range(HR // LANES)]
        rep = lax.iota(jnp.int32, LANES) // R
        cidx = [rep + k * R for k in range(HR // LANES)]

        @plsc.parallel_loop(0, b_per_w // LANES, unroll=8)
        def quad_body(i):
            v = idx_v[pl.ds(i * LANES, LANES)]
            q = (
                (v >= V4).astype(jnp.int32)
                + (v >= 2 * V4).astype(jnp.int32)
                + (v >= 3 * V4).astype(jnp.int32)
            )
            idx4_v[pl.ds(i * LANES, LANES)] = (v - q * V4) * R + q

        for b in range(NBUF):
            pltpu.async_copy(
                table_hbm.at[idx4_v.at[pl.ds(b * C, C)]], in_vs[b], gsems[b]
            )

        def outer(i, _):
            for b in range(NBUF):
                ch = i * NBUF + b
                in_v, out_v = in_vs[b], out_vs[b]
                pltpu.make_async_copy(
                    table_hbm.at[pl.ds(0, C)], in_v, gsems[b]
                ).wait()

                @pl.when(i >= 1)
                def _():
                    pltpu.make_async_copy(
                        out_v, out_hbm.at[pl.ds(base, C)], ssems[b]
                    ).wait()

                @plsc.parallel_loop(0, C, unroll=4)
                def row_body(row):
                    rsp = jnp.full((LANES,), row, dtype=jnp.int32)
                    for k in range(HR // LANES):
                        g = plsc.load_gather(in_v, [rsp, cidx[k]])
                        out_v[row, pl.ds(LANES * k, LANES)] = g * w_vecs[k]

                pltpu.async_copy(
                    out_v, out_hbm.at[pl.ds(base + ch * C, C)], ssems[b]
                )

                @pl.when(ch + NBUF < n_chunks)
                def _():
                    pltpu.async_copy(
                        table_hbm.at[idx4_v.at[pl.ds((ch + NBUF) * C, C)]],
                        in_v,
                        gsems[b],
                    )
            return 0

        lax.fori_loop(0, n_outer, outer, 0)
        for b in range(NBUF):
            pltpu.make_async_copy(
                out_vs[b], out_hbm.at[pl.ds(base, C)], ssems[b]
            ).wait()

    return pl.kernel(
        body,
        out_type=jax.ShapeDtypeStruct((B, HR), jnp.float32),
        mesh=mesh,
        scratch_types=[
            pltpu.VMEM((b_per_w,), jnp.int32),
            pltpu.VMEM((b_per_w,), jnp.int32),
            pltpu.VMEM((C, H), jnp.float32),
            pltpu.VMEM((C, H), jnp.float32),
            pltpu.VMEM((C, HR), jnp.float32),
            pltpu.VMEM((C, HR), jnp.float32),
            pltpu.VMEM((HR,), jnp.float32),
            pltpu.SemaphoreType.DMA,
            pltpu.SemaphoreType.DMA,
            pltpu.SemaphoreType.DMA,
            pltpu.SemaphoreType.DMA,
        ],
        compiler_params=pltpu.CompilerParams(
            use_tc_tiling_on_sc=False, needs_layout_passes=False
        ),
    )


def kernel(input_ids, table, w_up):
    bs, l = input_ids.shape
    B = bs * l
    V = table.shape[0]
    info = plsc.get_sparse_core_info()
    nw = info.num_cores * info.num_subcores
    b_per_w = B // nw
    assert b_per_w * nw == B and b_per_w % (C * NBUF) == 0
    assert V % 8 == 0 and 3 * _quarter_width(V) < V
    ids2 = input_ids.T.reshape(B).astype(jnp.int32)
    tt = table.T
    V4 = _quarter_width(V)
    table_lin = _tc_quadpack(V)(tt).reshape(R * V4, H)
    out2 = _sc_kernel(B, b_per_w, info.num_cores, V4)(ids2, table_lin, w_up)
    return out2.reshape(l, bs, HR).transpose(1, 0, 2)


# speedup vs baseline: 2.0534x; 1.0002x over previous
import jax
import jax.numpy as jnp
from jax import lax
from jax.experimental import pallas as pl
from jax.experimental.pallas import tpu as pltpu
from jax.experimental.pallas import tpu_sc as plsc

H = 32
R = 4
HR = H * R
LANES = 16
C = 128
NBUF = 5
TBLK = 16384


def _quarter_width(V):
    return (V // R + 127) // 128 * 128


def _tc_quadpack(V):
    V4 = _quarter_width(V)
    W3 = V - 3 * V4
    grid = pl.cdiv(V4, TBLK)
    g3f = W3 // TBLK
    edge = W3 - g3f * TBLK

    def body(tt_ref, o_ref, xb0, xb1, xbe, sem):
        g = pl.program_id(0)
        xbufs = (xb0, xb1)

        def start(gi, xb):
            for q in range(R - 1):
                pltpu.make_async_copy(
                    tt_ref.at[:, pl.ds(q * V4 + gi * TBLK, TBLK)],
                    xb.at[pl.ds(q * H, H), :],
                    sem,
                ).start()

            @pl.when(gi < g3f)
            def _():
                pltpu.make_async_copy(
                    tt_ref.at[:, pl.ds(3 * V4 + gi * TBLK, TBLK)],
                    xb.at[pl.ds(3 * H, H), :],
                    sem,
                ).start()

            if edge:
                @pl.when(gi == g3f)
                def _():
                    pltpu.make_async_copy(
                        tt_ref.at[:, pl.ds(3 * V4 + g3f * TBLK, edge)],
                        xbe,
                        sem,
                    ).start()

        def wait_for(gi, xb):
            for q in range(R - 1):
                pltpu.make_async_copy(
                    tt_ref.at[:, pl.ds(0, TBLK)],
                    xb.at[pl.ds(q * H, H), :],
                    sem,
                ).wait()

            @pl.when(gi < g3f)
            def _():
                pltpu.make_async_copy(
                    tt_ref.at[:, pl.ds(0, TBLK)],
                    xb.at[pl.ds(3 * H, H), :],
                    sem,
                ).wait()

            if edge:
                @pl.when(gi == g3f)
                def _():
                    pltpu.make_async_copy(
                        tt_ref.at[:, pl.ds(3 * V4 + g3f * TBLK, edge)],
                        xbe,
                        sem,
                    ).wait()

        @pl.when(g == 0)
        def _():
            start(0, xbufs[0])

        def phase(par):
            @pl.when(lax.rem(g, 2) == par)
            def _():
                xb = xbufs[par]
                nxt = xbufs[1 - par]

                @pl.when(g + 1 < grid)
                def _():
                    start(g + 1, nxt)

                wait_for(g, xb)
                o_ref[...] = xb[...].T
                if edge:
                    @pl.when(g == g3f)
                    def _():
                        o_ref[pl.ds(0, edge), (R - 1) * H:] = xbe[...].T

        phase(0)
        phase(1)

    return pl.pallas_call(
        body,
        grid=(grid,),
        in_specs=[pl.BlockSpec(memory_space=pl.ANY)],
        out_specs=pl.BlockSpec((TBLK, HR), lambda g: (g, 0)),
        out_shape=jax.ShapeDtypeStruct((V4, HR), jnp.float32),
        scratch_shapes=[
            pltpu.VMEM((HR, TBLK), jnp.float32),
            pltpu.VMEM((HR, TBLK), jnp.float32),
            pltpu.VMEM((H, edge if edge else 128), jnp.float32),
            pltpu.SemaphoreType.DMA,
        ],
    )


def _sc_kernel(B, b_per_w, num_cores, V4):
    mesh = plsc.VectorSubcoreMesh(core_axis_name="c", subcore_axis_name="s")
    n_chunks = b_per_w // C
    n_outer = n_chunks // NBUF

    def body(ids_hbm, table_hbm, w_hbm, out_hbm, idx_v, idx4_v, w_v, *scr):
        in_vs = list(scr[0:NBUF])
        out_vs = list(scr[NBUF:2 * NBUF])
        gsems = list(scr[2 * NBUF:3 * NBUF])
        ssems = list(scr[3 * NBUF:4 * NBUF])
        wid = lax.axis_index("s") * num_cores + lax.axis_index("c")
        base = wid * b_per_w
        pltpu.sync_copy(
            ids_hbm.at[pl.ds(base, b_per_w)], idx_v.at[pl.ds(0, b_per_w)]
        )
        pltpu.sync_copy(w_hbm, w_v)
        w_vecs = [w_v[pl.ds(LANES * k, LANES)] for k in range(HR // LANES)]
        rep = lax.iota(jnp.int32, LANES) // R
        cidx = [rep + k * R for k in range(HR // LANES)]

        @plsc.parallel_loop(0, b_per_w // LANES, unroll=8)
        def quad_body(i):
            v = idx_v[pl.ds(i * LANES, LANES)]
            q = (
                (v >= V4).astype(jnp.int32)
                + (v >= 2 * V4).astype(jnp.int32)
                + (v >= 3 * V4).astype(jnp.int32)
            )
            idx4_v[pl.ds(i * LANES, LANES)] = (v - q * V4) * R + q

        for b in range(NBUF):
            pltpu.async_copy(
                table_hbm.at[idx4_v.at[pl.ds(b * C, C)]], in_vs[b], gsems[b]
            )

        def outer(i, _):
            for b in range(NBUF):
                ch = i * NBUF + b
                in_v, out_v = in_vs[b], out_vs[b]
                pltpu.make_async_copy(
                    table_hbm.at[pl.ds(0, C)], in_v, gsems[b]
                ).wait()

                @pl.when(i >= 1)
                def _():
                    pltpu.make_async_copy(
                        out_v, out_hbm.at[pl.ds(base, C)], ssems[b]
                    ).wait()

                @plsc.parallel_loop(0, C, unroll=4)
                def row_body(row):
                    rsp = jnp.full((LANES,), row, dtype=jnp.int32)
                    for k in range(HR // LANES):
                        g = plsc.load_gather(in_v, [rsp, cidx[k]])
                        out_v[row, pl.ds(LANES * k, LANES)] = g * w_vecs[k]

                pltpu.async_copy(
                    out_v, out_hbm.at[pl.ds(base + ch * C, C)], ssems[b]
                )

                @pl.when(ch + NBUF < n_chunks)
                def _():
                    pltpu.async_copy(
                        table_hbm.at[idx4_v.at[pl.ds((ch + NBUF) * C, C)]],
                        in_v,
                        gsems[b],
                    )
            return 0

        lax.fori_loop(0, n_outer, outer, 0)
        for b in range(NBUF):
            pltpu.make_async_copy(
                out_vs[b], out_hbm.at[pl.ds(base, C)], ssems[b]
            ).wait()

    return pl.kernel(
        body,
        out_type=jax.ShapeDtypeStruct((B, HR), jnp.float32),
        mesh=mesh,
        scratch_types=(
            [
                pltpu.VMEM((b_per_w,), jnp.int32),
                pltpu.VMEM((b_per_w,), jnp.int32),
                pltpu.VMEM((HR,), jnp.float32),
            ]
            + [pltpu.VMEM((C, H), jnp.float32) for _ in range(NBUF)]
            + [pltpu.VMEM((C, HR), jnp.float32) for _ in range(NBUF)]
            + [pltpu.SemaphoreType.DMA for _ in range(2 * NBUF)]
        ),
        compiler_params=pltpu.CompilerParams(
            use_tc_tiling_on_sc=False, needs_layout_passes=False
        ),
    )


def kernel(input_ids, table, w_up):
    bs, l = input_ids.shape
    B = bs * l
    V = table.shape[0]
    info = plsc.get_sparse_core_info()
    nw = info.num_cores * info.num_subcores
    b_per_w = B // nw
    assert b_per_w * nw == B and b_per_w % (C * NBUF) == 0
    assert V % 8 == 0 and 3 * _quarter_width(V) < V
    ids2 = input_ids.T.reshape(B).astype(jnp.int32)
    tt = table.T
    V4 = _quarter_width(V)
    table_lin = _tc_quadpack(V)(tt).reshape(R * V4, H)
    out2 = _sc_kernel(B, b_per_w, info.num_cores, V4)(ids2, table_lin, w_up)
    return out2.reshape(l, bs, HR).transpose(1, 0, 2)
